# Initial kernel scaffold; baseline (speedup 1.0000x reference)
import jax
import jax.numpy as jnp
from jax.experimental import pallas as pl


def kernel(x, W, b):
    raise NotImplementedError("write your pallas kernel here")


# speedup vs baseline: 2.4400x; 2.4400x over previous
import functools

import jax
import jax.numpy as jnp
from jax.experimental import pallas as pl
from jax.experimental.pallas import tpu as pltpu

_TILE = 1024


def _router_body(x_ref, wt_ref, b_ref, w_out_ref, i_out_ref):
    scores = jax.lax.dot_general(
        x_ref[...], wt_ref[...],
        (((1,), (0,)), ((), ())),
        preferred_element_type=jnp.float32,
    )
    scores = scores + b_ref[...]
    m = jnp.max(scores, axis=-1, keepdims=True)
    e = jnp.exp(scores - m)
    s = e / jnp.sum(e, axis=-1, keepdims=True)
    n = s.shape[-1]
    iota = jax.lax.broadcasted_iota(jnp.int32, s.shape, 1)
    m1 = jnp.max(s, axis=-1, keepdims=True)
    i1 = jnp.min(jnp.where(s == m1, iota, n), axis=-1, keepdims=True)
    s2 = jnp.where(iota == i1, -jnp.inf, s)
    m2 = jnp.max(s2, axis=-1, keepdims=True)
    i2 = jnp.min(jnp.where(s2 == m2, iota, n), axis=-1, keepdims=True)
    w_out_ref[...] = jnp.concatenate([m1, m2], axis=1)
    i_out_ref[...] = jnp.concatenate([i1, i2], axis=1)


@functools.partial(jax.jit, static_argnames=("interpret",))
def kernel(x, W, b, interpret=False):
    ntok, dim = x.shape
    nexp = W.shape[0]
    wt = W.T
    b2 = b.reshape(1, nexp)
    grid = (ntok // _TILE,)
    weights, idx = pl.pallas_call(
        _router_body,
        grid=grid,
        in_specs=[
            pl.BlockSpec((_TILE, dim), lambda i: (i, 0)),
            pl.BlockSpec((dim, nexp), lambda i: (0, 0)),
            pl.BlockSpec((1, nexp), lambda i: (0, 0)),
        ],
        out_specs=[
            pl.BlockSpec((_TILE, 2), lambda i: (i, 0)),
            pl.BlockSpec((_TILE, 2), lambda i: (i, 0)),
        ],
        out_shape=[
            jax.ShapeDtypeStruct((ntok, 2), jnp.float32),
            jax.ShapeDtypeStruct((ntok, 2), jnp.int32),
        ],
        interpret=interpret,
    )(x, wt, b2)
    return weights, idx
